# baseline (speedup 1.0000x reference)
import functools

import jax
import jax.numpy as jnp
from jax import lax
from jax.experimental import pallas as pl
from jax.experimental.pallas import tpu as pltpu
from jax.experimental.pallas import tpu_sc as plsc

_B = 16384
_DIM = 64
_NENTITY = 1000000
_NATTR = 500
_NE2 = _NENTITY // 2
_NA2 = _NATTR // 2
_NW = 32
_BPW = _B // _NW
_CHUNK = 16
_NCHK = _BPW // _CHUNK
_NBUF = 4


def _q2b_body(ent_hbm, attr_hbm, ent2, attr2, off2,
              out_hbm, eidx_v, aidx_v, a_v, o_v, e0, e1, e2, e3, out_v,
              s0, s1, s2, s3, sem_t):
    wid = lax.axis_index("s") * 2 + lax.axis_index("c")
    base = wid * _BPW
    bufs = (e0, e1, e2, e3)
    sems = (s0, s1, s2, s3)

    pltpu.sync_copy(ent_hbm.at[pl.ds(base, _BPW)], eidx_v)
    pltpu.sync_copy(attr_hbm.at[pl.ds(base, _BPW)], aidx_v)

    ct_a = pltpu.async_copy(attr2, a_v, sem_t)
    ct_o = pltpu.async_copy(off2, o_v, sem_t)

    def issue(c, b):
        idx16 = eidx_v[pl.ds(c * _CHUNK, _CHUNK)]
        erow = lax.shift_right_logical(idx16, 1)
        pltpu.async_copy(ent2.at[erow], bufs[b], sems[b])

    for b in range(_NBUF - 1):
        issue(b, b)
    ct_a.wait()
    ct_o.wait()

    iot = lax.iota(jnp.int32, 16)

    def step(it, _):
        for b in range(_NBUF):
            c = it * _NBUF + b
            nxt = c + _NBUF - 1

            @pl.when(nxt < _NCHK)
            def _():
                issue(nxt, (b + _NBUF - 1) % _NBUF)

            pltpu.make_async_copy(ent2.at[pl.ds(0, _CHUNK)], bufs[b],
                                  sems[b]).wait()
            eidx16 = eidx_v[pl.ds(c * _CHUNK, _CHUNK)]
            ecol = jnp.bitwise_and(eidx16, 1) * _DIM
            aidx16 = aidx_v[pl.ds(c * _CHUNK, _CHUNK)]
            arow = lax.shift_right_logical(aidx16, 1)
            acol = jnp.bitwise_and(aidx16, 1) * _DIM
            acc = jnp.zeros((16,), jnp.float32)
            accn = jnp.zeros((16,), jnp.float32)
            for d in range(_DIM):
                dv = jnp.full((16,), d, jnp.int32)
                e_d = plsc.load_gather(bufs[b], [iot, ecol + dv])
                a_d = plsc.load_gather(a_v, [arow, acol + dv])
                o_d = plsc.load_gather(o_v, [arow, acol + dv])
                acc = acc + jnp.maximum(o_d - jnp.abs(e_d - a_d), 0.0)
                accn = accn + jnp.abs(o_d)
            out_v[pl.ds(c * _CHUNK, _CHUNK)] = 1.0 - acc / accn
        return ()

    lax.fori_loop(0, _NCHK // _NBUF, step, ())

    pltpu.sync_copy(out_v, out_hbm.at[pl.ds(base, _BPW)])


@functools.cache
def _build():
    mesh = plsc.VectorSubcoreMesh(core_axis_name="c", subcore_axis_name="s")
    return pl.kernel(
        _q2b_body,
        mesh=mesh,
        out_type=jax.ShapeDtypeStruct((_B,), jnp.float32),
        scratch_types=[
            pltpu.VMEM((_BPW,), jnp.int32),
            pltpu.VMEM((_BPW,), jnp.int32),
            pltpu.VMEM((_NA2, 2 * _DIM), jnp.float32),
            pltpu.VMEM((_NA2, 2 * _DIM), jnp.float32),
            pltpu.VMEM((_CHUNK, 2 * _DIM), jnp.float32),
            pltpu.VMEM((_CHUNK, 2 * _DIM), jnp.float32),
            pltpu.VMEM((_CHUNK, 2 * _DIM), jnp.float32),
            pltpu.VMEM((_CHUNK, 2 * _DIM), jnp.float32),
            pltpu.VMEM((_BPW,), jnp.float32),
            pltpu.SemaphoreType.DMA,
            pltpu.SemaphoreType.DMA,
            pltpu.SemaphoreType.DMA,
            pltpu.SemaphoreType.DMA,
            pltpu.SemaphoreType.DMA,
        ],
        compiler_params=pltpu.CompilerParams(needs_layout_passes=False),
    )


def kernel(entities, attributes, ent_emb, attr_emb, offset_attr_emb):
    ent2 = ent_emb.reshape(_NE2, 2 * _DIM)
    attr2 = attr_emb.reshape(_NA2, 2 * _DIM)
    off2 = offset_attr_emb.reshape(_NA2, 2 * _DIM)
    return _build()(entities, attributes, ent2, attr2, off2)
